# Initial kernel scaffold; baseline (speedup 1.0000x reference)
import jax
import jax.numpy as jnp
from jax.experimental import pallas as pl

B = 32
SIZE = 32
P = SIZE * SIZE
D = 2048
LEVELS = 256
D_TILE = 1024
D_TILES = D // D_TILE


def _tc_body(x_ref, pos_ref, val_ref, out_ref):
    flat = x_ref[0].reshape(P, 1)
    idx = jnp.clip(jnp.round(flat * (LEVELS - 1)), 0, LEVELS - 1).astype(jnp.int32)
    iota = jax.lax.broadcasted_iota(jnp.int32, (P, LEVELS), 1)
    oh = jnp.where(iota == idx, 1.0, 0.0).astype(jnp.bfloat16)
    valb = val_ref[...].astype(jnp.bfloat16)
    vx = jax.lax.dot(oh, valb, preferred_element_type=jnp.float32)
    posf = pos_ref[...].astype(jnp.float32)
    counts = jnp.sum(posf + (1.0 - 2.0 * posf) * vx, axis=0)
    out_ref[0, :] = (2.0 * counts > float(P)).astype(jnp.int32)


@jax.jit
def kernel(x, position_weight, value_weight):
    grid = (D_TILES, B)
    return pl.pallas_call(
        _tc_body,
        grid=grid,
        in_specs=[
            pl.BlockSpec((1, SIZE, SIZE), lambda dt, b: (b, 0, 0)),
            pl.BlockSpec((P, D_TILE), lambda dt, b: (0, dt)),
            pl.BlockSpec((LEVELS, D_TILE), lambda dt, b: (0, dt)),
        ],
        out_specs=pl.BlockSpec((1, D_TILE), lambda dt, b: (b, dt)),
        out_shape=jax.ShapeDtypeStruct((B, D), jnp.int32),
    )(x, position_weight, value_weight)


# speedup vs baseline: 2.6272x; 2.6272x over previous
import jax
import jax.numpy as jnp
from jax.experimental import pallas as pl

B = 32
SIZE = 32
P = SIZE * SIZE
D = 2048
LEVELS = 256
D_TILE = 1024
D_TILES = D // D_TILE


def _tc_body(x_ref, pos_ref, val_ref, out_ref):
    flat = x_ref[0]
    idx = jnp.clip(jnp.round(flat * (LEVELS - 1)), 0, LEVELS - 1).astype(jnp.int32)
    iota = jax.lax.broadcasted_iota(jnp.int32, (P, LEVELS), 1)
    oh = jnp.where(iota == idx, 1.0, 0.0).astype(jnp.bfloat16)
    valb = val_ref[...].astype(jnp.bfloat16)
    vx = jax.lax.dot(oh, valb, preferred_element_type=jnp.float32)
    posf = pos_ref[...].astype(jnp.float32)
    counts = jnp.sum(posf + (1.0 - 2.0 * posf) * vx, axis=0)
    out_ref[0, 0, :] = (2.0 * counts > float(P)).astype(jnp.int32)


@jax.jit
def kernel(x, position_weight, value_weight):
    grid = (D_TILES, B)
    xc = x.reshape(B, P, 1)
    return pl.pallas_call(
        _tc_body,
        grid=grid,
        in_specs=[
            pl.BlockSpec((1, P, 1), lambda dt, b: (b, 0, 0)),
            pl.BlockSpec((P, D_TILE), lambda dt, b: (0, dt)),
            pl.BlockSpec((LEVELS, D_TILE), lambda dt, b: (0, dt)),
        ],
        out_specs=pl.BlockSpec((1, 1, D_TILE), lambda dt, b: (b, 0, dt)),
        out_shape=jax.ShapeDtypeStruct((B, 1, D), jnp.int32),
    )(xc, position_weight, value_weight).reshape(B, D)
